# baseline (speedup 1.0000x reference)
import functools

import jax
import jax.numpy as jnp
from jax import lax
from jax.experimental import pallas as pl
from jax.experimental.pallas import tpu as pltpu
from jax.experimental.pallas import tpu_sc as plsc

N = 10000
NP = 10240
E = 160000
NB = 80
EPT = NB * 128
EP = 16 * EPT
RPT = NP // 16
T = 512
NT = NP // T

_f32 = jnp.float32


def _sc_mesh():
    return plsc.VectorSubcoreMesh(
        core_axis_name="c", subcore_axis_name="s", num_cores=2, num_subcores=16
    )


def _make_deg():
    @functools.partial(
        pl.kernel,
        out_type=jax.ShapeDtypeStruct((NP,), _f32),
        mesh=_sc_mesh(),
        scratch_types=[
            pltpu.VMEM((NB, 128), jnp.int32),
            pltpu.VMEM((128,), _f32),
            pltpu.VMEM_SHARED((NP,), _f32),
        ],
    )
    def deg_kernel(dst_hbm, ones_hbm, zeros1_hbm, cnt_hbm, dst_v, ones_v, acc_sh):
        c = lax.axis_index("c")
        s = lax.axis_index("s")
        pltpu.sync_copy(dst_hbm.at[s], dst_v)
        pltpu.sync_copy(ones_hbm, ones_v)
        pltpu.sync_copy(zeros1_hbm, acc_sh.at[pl.ds(s * RPT, RPT)])
        plsc.subcore_barrier()

        def body(b, carry):
            pltpu.sync_copy(ones_v, acc_sh.at[dst_v.at[b]], add=True)
            return carry

        lax.fori_loop(0, NB, body, 0)
        plsc.subcore_barrier()

        @pl.when(c == 0)
        def _():
            pltpu.sync_copy(
                acc_sh.at[pl.ds(s * RPT, RPT)], cnt_hbm.at[pl.ds(s * RPT, RPT)]
            )

    return deg_kernel


def _make_agg(C):
    CC = C // 2

    NG = NB // 16

    if True:
        @functools.partial(
            pl.kernel,
            out_type=jax.ShapeDtypeStruct((C, NP, 128), _f32),
            mesh=_sc_mesh(),
            scratch_types=[
                pltpu.VMEM((2, 16, 128), jnp.int32),
                pltpu.VMEM((NB, 128), jnp.int32),
                pltpu.VMEM((2, 128, 128), _f32),
                pltpu.VMEM_SHARED((NP, 128), _f32),
                pltpu.SemaphoreType.DMA,
                pltpu.SemaphoreType.DMA,
                pltpu.SemaphoreType.DMA,
                pltpu.SemaphoreType.DMA,
                pltpu.SemaphoreType.DMA,
            ],
        )
        def agg_kernel(g_hbm, srcall_hbm, dst_hbm, zeros_hbm, s_hbm,
                       src_g, dst_v, rows_v, acc_sh,
                       sem_a, sem_b, sem_s, sem_s0, sem_s1):
            c = lax.axis_index("c")
            s = lax.axis_index("s")
            pltpu.sync_copy(dst_hbm.at[s], dst_v)
            pltpu.sync_copy(zeros_hbm, acc_sh.at[pl.ds(s * RPT, RPT)])
            plsc.subcore_barrier()
            for j in range(CC):
                cc = c * CC + j
                pltpu.sync_copy(srcall_hbm.at[cc, s, pl.ds(0, 16)], src_g.at[0])

                def group_body(g, carry):
                    gp = g % 2

                    @pl.when(g > 0)
                    def _():
                        pltpu.make_async_copy(
                            srcall_hbm.at[cc, s, pl.ds(g * 16, 16)],
                            src_g.at[gp], sem_s).wait()

                    @pl.when(g < NG - 1)
                    def _():
                        pltpu.async_copy(
                            srcall_hbm.at[cc, s, pl.ds((g + 1) * 16, 16)],
                            src_g.at[1 - gp], sem_s)

                    pltpu.async_copy(
                        g_hbm.at[src_g.at[gp, 0]], rows_v.at[0], sem_a)

                    def pair(i, carry2):
                        b0 = 2 * i
                        b1 = b0 + 1
                        base = g * 16
                        pltpu.async_copy(
                            g_hbm.at[src_g.at[gp, b1]], rows_v.at[1], sem_b)
                        pltpu.make_async_copy(
                            g_hbm.at[src_g.at[gp, b0]], rows_v.at[0],
                            sem_a).wait()
                        pltpu.async_copy(
                            rows_v.at[0], acc_sh.at[dst_v.at[base + b0]],
                            sem_s0, add=True)
                        pltpu.make_async_copy(
                            g_hbm.at[src_g.at[gp, b1]], rows_v.at[1],
                            sem_b).wait()
                        pltpu.async_copy(
                            rows_v.at[1], acc_sh.at[dst_v.at[base + b1]],
                            sem_s1, add=True)
                        pltpu.make_async_copy(
                            rows_v.at[0], acc_sh.at[dst_v.at[base + b0]],
                            sem_s0).wait()

                        @pl.when(i < 7)
                        def _():
                            pltpu.async_copy(
                                g_hbm.at[src_g.at[gp, b0 + 2]], rows_v.at[0],
                                sem_a)

                        pltpu.make_async_copy(
                            rows_v.at[1], acc_sh.at[dst_v.at[base + b1]],
                            sem_s1).wait()
                        return carry2

                    lax.fori_loop(0, 8, pair, 0)
                    return carry

                lax.fori_loop(0, NG, group_body, 0)
                plsc.subcore_barrier()
                pltpu.sync_copy(
                    acc_sh.at[pl.ds(s * RPT, RPT)],
                    s_hbm.at[cc, pl.ds(s * RPT, RPT)],
                )
                if j + 1 < CC:
                    pltpu.sync_copy(zeros_hbm, acc_sh.at[pl.ds(s * RPT, RPT)])
                    plsc.subcore_barrier()

    return agg_kernel


_deg = _make_deg()
_agg2 = _make_agg(2)
_agg4 = _make_agg(4)


def _prep(cnt2d, x_pad):
    def body(cnt_ref, x_ref, g_ref, dr_ref):
        dinv = lax.rsqrt(cnt_ref[...] + 1.0)
        dinv_rep = jnp.broadcast_to(dinv, (T, 128))
        g_ref[0] = x_ref[...] * dinv_rep

        @pl.when(pl.program_id(1) == 0)
        def _():
            dr_ref[...] = dinv_rep

    return pl.pallas_call(
        body,
        grid=(NT, 2),
        in_specs=[
            pl.BlockSpec((T, 1), lambda i, c: (i, 0)),
            pl.BlockSpec((T, 128), lambda i, c: (i, c)),
        ],
        out_specs=[
            pl.BlockSpec((1, T, 128), lambda i, c: (c, i, 0)),
            pl.BlockSpec((T, 128), lambda i, c: (i, 0)),
        ],
        out_shape=[
            jax.ShapeDtypeStruct((2, NP, 128), _f32),
            jax.ShapeDtypeStruct((NP, 128), _f32),
        ],
    )(cnt2d, x_pad)


def _layer(s_arr, g_arr, dinv_rep, W, b, Cin):
    Cout = 4

    def body(s_ref, g_ref, d_ref, w_ref, b_ref, o_ref):
        d = d_ref[...]
        acc = jnp.zeros((T, 512), _f32)
        for ci in range(Cin):
            u = (s_ref[ci] + g_ref[ci]) * d
            acc += jnp.dot(u, w_ref[pl.ds(ci * 128, 128), :],
                           preferred_element_type=_f32)
        x = jnp.maximum(acc + b_ref[...], 0.0)
        for co in range(Cout):
            o_ref[co] = x[:, co * 128:(co + 1) * 128] * d

    return pl.pallas_call(
        body,
        grid=(NT,),
        in_specs=[
            pl.BlockSpec((Cin, T, 128), lambda i: (0, i, 0)),
            pl.BlockSpec((Cin, T, 128), lambda i: (0, i, 0)),
            pl.BlockSpec((T, 128), lambda i: (i, 0)),
            pl.BlockSpec((Cin * 128, 512), lambda i: (0, 0)),
            pl.BlockSpec((1, 512), lambda i: (0, 0)),
        ],
        out_specs=pl.BlockSpec((Cout, T, 128), lambda i: (0, i, 0)),
        out_shape=jax.ShapeDtypeStruct((Cout, NP, 128), _f32),
    )(s_arr, g_arr, dinv_rep, W, b.reshape(1, 512))


def _head(s_arr, g_arr, dinv_rep, W3, b3, Wout, bout):
    def body(s_ref, g_ref, d_ref, w3_ref, b3_ref, wo_ref, bo_ref, o_ref):
        d = d_ref[...]
        acc = jnp.zeros((T, 512), _f32)
        for ci in range(4):
            u = (s_ref[ci] + g_ref[ci]) * d
            acc += jnp.dot(u, w3_ref[pl.ds(ci * 128, 128), :],
                           preferred_element_type=_f32)
        x4 = jnp.maximum(acc + b3_ref[...], 0.0)
        o_ref[...] = jnp.dot(x4, wo_ref[...], preferred_element_type=_f32) + bo_ref[...]

    return pl.pallas_call(
        body,
        grid=(NT,),
        in_specs=[
            pl.BlockSpec((4, T, 128), lambda i: (0, i, 0)),
            pl.BlockSpec((4, T, 128), lambda i: (0, i, 0)),
            pl.BlockSpec((T, 128), lambda i: (i, 0)),
            pl.BlockSpec((512, 512), lambda i: (0, 0)),
            pl.BlockSpec((1, 512), lambda i: (0, 0)),
            pl.BlockSpec((512, 512), lambda i: (0, 0)),
            pl.BlockSpec((1, 512), lambda i: (0, 0)),
        ],
        out_specs=pl.BlockSpec((T, 512), lambda i: (i, 0)),
        out_shape=jax.ShapeDtypeStruct((NP, 512), _f32),
    )(s_arr, g_arr, dinv_rep, W3, b3.reshape(1, 512), Wout, bout.reshape(1, 512))


def kernel(x_node, edge_index, W1, b1, W2, b2, W3, b3, Wout, bout):
    src = edge_index[0].astype(jnp.int32)
    dst = edge_index[1].astype(jnp.int32)
    pad = EP - E
    pad_src = (jnp.arange(pad, dtype=jnp.int32) * 97) % N
    pad_dst = N + jnp.arange(pad, dtype=jnp.int32) % (NP - N)
    srcp = jnp.concatenate([src, pad_src]).reshape(16, EPT)
    dstp = jnp.concatenate([dst, pad_dst]).reshape(16, NB, 128)
    src2 = (srcp[None] + (jnp.arange(2, dtype=jnp.int32) * NP)[:, None, None]
            ).reshape(2, 16, NB, 128)
    src4 = (srcp[None] + (jnp.arange(4, dtype=jnp.int32) * NP)[:, None, None]
            ).reshape(4, 16, NB, 128)
    zeros2d = jnp.zeros((RPT, 128), _f32)
    zeros1d = jnp.zeros((RPT,), _f32)
    ones1d = jnp.ones((128,), _f32)
    x_pad = jnp.pad(x_node, ((0, NP - N), (0, 0)))

    cnt = _deg(dstp, ones1d, zeros1d)
    g1, dinv_rep = _prep(cnt.reshape(NP, 1), x_pad)
    s1 = _agg2(g1.reshape(2 * NP, 128), src2, dstp, zeros2d)
    g2 = _layer(s1, g1, dinv_rep, W1, b1, Cin=2)
    s2 = _agg4(g2.reshape(4 * NP, 128), src4, dstp, zeros2d)
    g3 = _layer(s2, g2, dinv_rep, W2, b2, Cin=4)
    s3 = _agg4(g3.reshape(4 * NP, 128), src4, dstp, zeros2d)
    out = _head(s3, g3, dinv_rep, W3, b3, Wout, bout)
    return out[:N]


# speedup vs baseline: 1.2164x; 1.2164x over previous
import functools

import jax
import jax.numpy as jnp
from jax import lax
from jax.experimental import pallas as pl
from jax.experimental.pallas import tpu as pltpu
from jax.experimental.pallas import tpu_sc as plsc

N = 10000
NP = 10240
E = 160000
NB = 80
EPT = NB * 128
EP = 16 * EPT
RPT = NP // 16
T = 512
NT = NP // T

_f32 = jnp.float32


def _sc_mesh():
    return plsc.VectorSubcoreMesh(
        core_axis_name="c", subcore_axis_name="s", num_cores=2, num_subcores=16
    )


def _make_deg():
    @functools.partial(
        pl.kernel,
        out_type=jax.ShapeDtypeStruct((NP,), _f32),
        mesh=_sc_mesh(),
        scratch_types=[
            pltpu.VMEM((NB, 128), jnp.int32),
            pltpu.VMEM((128,), _f32),
            pltpu.VMEM_SHARED((NP,), _f32),
        ],
    )
    def deg_kernel(dst_hbm, ones_hbm, zeros1_hbm, cnt_hbm, dst_v, ones_v, acc_sh):
        c = lax.axis_index("c")
        s = lax.axis_index("s")
        pltpu.sync_copy(dst_hbm.at[s], dst_v)
        pltpu.sync_copy(ones_hbm, ones_v)
        pltpu.sync_copy(zeros1_hbm, acc_sh.at[pl.ds(s * RPT, RPT)])
        plsc.subcore_barrier()

        def body(b, carry):
            pltpu.sync_copy(ones_v, acc_sh.at[dst_v.at[b]], add=True)
            return carry

        lax.fori_loop(0, NB, body, 0)
        plsc.subcore_barrier()

        @pl.when(c == 0)
        def _():
            pltpu.sync_copy(
                acc_sh.at[pl.ds(s * RPT, RPT)], cnt_hbm.at[pl.ds(s * RPT, RPT)]
            )

    return deg_kernel


def _make_agg(C):
    CC = C // 2

    NG = NB // 16

    if True:
        @functools.partial(
            pl.kernel,
            out_type=jax.ShapeDtypeStruct((C, NP, 128), _f32),
            mesh=_sc_mesh(),
            scratch_types=[
                pltpu.VMEM((2, 16, 128), jnp.int32),
                pltpu.VMEM((NB, 128), jnp.int32),
                pltpu.VMEM((2, 128, 128), _f32),
                pltpu.VMEM_SHARED((NP, 128), _f32),
                pltpu.SemaphoreType.DMA,
                pltpu.SemaphoreType.DMA,
                pltpu.SemaphoreType.DMA,
            ],
        )
        def agg_kernel(g_hbm, srcall_hbm, dst_hbm, zeros_hbm, s_hbm,
                       src_g, dst_v, rows_v, acc_sh, sem_a, sem_b, sem_s):
            c = lax.axis_index("c")
            s = lax.axis_index("s")
            pltpu.sync_copy(dst_hbm.at[s], dst_v)
            pltpu.sync_copy(zeros_hbm, acc_sh.at[pl.ds(s * RPT, RPT)])
            plsc.subcore_barrier()
            for j in range(CC):
                cc = c * CC + j
                pltpu.sync_copy(srcall_hbm.at[cc, s, pl.ds(0, 16)], src_g.at[0])

                def group_body(g, carry):
                    gp = g % 2

                    @pl.when(g > 0)
                    def _():
                        pltpu.make_async_copy(
                            srcall_hbm.at[cc, s, pl.ds(g * 16, 16)],
                            src_g.at[gp], sem_s).wait()

                    @pl.when(g < NG - 1)
                    def _():
                        pltpu.async_copy(
                            srcall_hbm.at[cc, s, pl.ds((g + 1) * 16, 16)],
                            src_g.at[1 - gp], sem_s)

                    pltpu.async_copy(
                        g_hbm.at[src_g.at[gp, 0]], rows_v.at[0], sem_a)

                    def pair(i, carry2):
                        b0 = 2 * i
                        b1 = b0 + 1
                        base = g * 16
                        pltpu.async_copy(
                            g_hbm.at[src_g.at[gp, b1]], rows_v.at[1], sem_b)
                        pltpu.make_async_copy(
                            g_hbm.at[src_g.at[gp, b0]], rows_v.at[0],
                            sem_a).wait()
                        pltpu.sync_copy(
                            rows_v.at[0], acc_sh.at[dst_v.at[base + b0]],
                            add=True)

                        @pl.when(i < 7)
                        def _():
                            pltpu.async_copy(
                                g_hbm.at[src_g.at[gp, b0 + 2]], rows_v.at[0],
                                sem_a)

                        pltpu.make_async_copy(
                            g_hbm.at[src_g.at[gp, b1]], rows_v.at[1],
                            sem_b).wait()
                        pltpu.sync_copy(
                            rows_v.at[1], acc_sh.at[dst_v.at[base + b1]],
                            add=True)
                        return carry2

                    lax.fori_loop(0, 8, pair, 0)
                    return carry

                lax.fori_loop(0, NG, group_body, 0)
                plsc.subcore_barrier()
                pltpu.sync_copy(
                    acc_sh.at[pl.ds(s * RPT, RPT)],
                    s_hbm.at[cc, pl.ds(s * RPT, RPT)],
                )
                if j + 1 < CC:
                    pltpu.sync_copy(zeros_hbm, acc_sh.at[pl.ds(s * RPT, RPT)])
                    plsc.subcore_barrier()

    return agg_kernel


_deg = _make_deg()
_agg2 = _make_agg(2)
_agg4 = _make_agg(4)


def _prep(cnt2d, x_pad):
    def body(cnt_ref, x_ref, g_ref, dr_ref):
        dinv = lax.rsqrt(cnt_ref[...] + 1.0)
        dinv_rep = jnp.broadcast_to(dinv, (T, 128))
        g_ref[0] = x_ref[...] * dinv_rep

        @pl.when(pl.program_id(1) == 0)
        def _():
            dr_ref[...] = dinv_rep

    return pl.pallas_call(
        body,
        grid=(NT, 2),
        in_specs=[
            pl.BlockSpec((T, 1), lambda i, c: (i, 0)),
            pl.BlockSpec((T, 128), lambda i, c: (i, c)),
        ],
        out_specs=[
            pl.BlockSpec((1, T, 128), lambda i, c: (c, i, 0)),
            pl.BlockSpec((T, 128), lambda i, c: (i, 0)),
        ],
        out_shape=[
            jax.ShapeDtypeStruct((2, NP, 128), _f32),
            jax.ShapeDtypeStruct((NP, 128), _f32),
        ],
    )(cnt2d, x_pad)


def _layer(s_arr, g_arr, dinv_rep, W, b, Cin):
    Cout = 4

    def body(s_ref, g_ref, d_ref, w_ref, b_ref, o_ref):
        d = d_ref[...]
        acc = jnp.zeros((T, 512), _f32)
        for ci in range(Cin):
            u = ((s_ref[ci] + g_ref[ci]) * d).astype(jnp.bfloat16)
            acc += jnp.dot(u, w_ref[pl.ds(ci * 128, 128), :],
                           preferred_element_type=_f32)
        x = jnp.maximum(acc + b_ref[...], 0.0)
        for co in range(Cout):
            o_ref[co] = x[:, co * 128:(co + 1) * 128] * d

    return pl.pallas_call(
        body,
        grid=(NT,),
        in_specs=[
            pl.BlockSpec((Cin, T, 128), lambda i: (0, i, 0)),
            pl.BlockSpec((Cin, T, 128), lambda i: (0, i, 0)),
            pl.BlockSpec((T, 128), lambda i: (i, 0)),
            pl.BlockSpec((Cin * 128, 512), lambda i: (0, 0)),
            pl.BlockSpec((1, 512), lambda i: (0, 0)),
        ],
        out_specs=pl.BlockSpec((Cout, T, 128), lambda i: (0, i, 0)),
        out_shape=jax.ShapeDtypeStruct((Cout, NP, 128), _f32),
    )(s_arr, g_arr, dinv_rep, W, b.reshape(1, 512))


def _head(s_arr, g_arr, dinv_rep, W3, b3, Wout, bout):
    def body(s_ref, g_ref, d_ref, w3_ref, b3_ref, wo_ref, bo_ref, o_ref):
        d = d_ref[...]
        acc = jnp.zeros((T, 512), _f32)
        for ci in range(4):
            u = ((s_ref[ci] + g_ref[ci]) * d).astype(jnp.bfloat16)
            acc += jnp.dot(u, w3_ref[pl.ds(ci * 128, 128), :],
                           preferred_element_type=_f32)
        x4 = jnp.maximum(acc + b3_ref[...], 0.0).astype(jnp.bfloat16)
        o_ref[...] = jnp.dot(x4, wo_ref[...], preferred_element_type=_f32) + bo_ref[...]

    return pl.pallas_call(
        body,
        grid=(NT,),
        in_specs=[
            pl.BlockSpec((4, T, 128), lambda i: (0, i, 0)),
            pl.BlockSpec((4, T, 128), lambda i: (0, i, 0)),
            pl.BlockSpec((T, 128), lambda i: (i, 0)),
            pl.BlockSpec((512, 512), lambda i: (0, 0)),
            pl.BlockSpec((1, 512), lambda i: (0, 0)),
            pl.BlockSpec((512, 512), lambda i: (0, 0)),
            pl.BlockSpec((1, 512), lambda i: (0, 0)),
        ],
        out_specs=pl.BlockSpec((T, 512), lambda i: (i, 0)),
        out_shape=jax.ShapeDtypeStruct((NP, 512), _f32),
    )(s_arr, g_arr, dinv_rep, W3, b3.reshape(1, 512), Wout, bout.reshape(1, 512))


def kernel(x_node, edge_index, W1, b1, W2, b2, W3, b3, Wout, bout):
    src = edge_index[0].astype(jnp.int32)
    dst = edge_index[1].astype(jnp.int32)
    pad = EP - E
    pad_src = (jnp.arange(pad, dtype=jnp.int32) * 97) % N
    pad_dst = N + jnp.arange(pad, dtype=jnp.int32) % (NP - N)
    srcp = jnp.concatenate([src, pad_src]).reshape(16, EPT)
    dstp = jnp.concatenate([dst, pad_dst]).reshape(16, NB, 128)
    src2 = (srcp[None] + (jnp.arange(2, dtype=jnp.int32) * NP)[:, None, None]
            ).reshape(2, 16, NB, 128)
    src4 = (srcp[None] + (jnp.arange(4, dtype=jnp.int32) * NP)[:, None, None]
            ).reshape(4, 16, NB, 128)
    zeros2d = jnp.zeros((RPT, 128), _f32)
    zeros1d = jnp.zeros((RPT,), _f32)
    ones1d = jnp.ones((128,), _f32)
    x_pad = jnp.pad(x_node, ((0, NP - N), (0, 0)))

    bf16 = jnp.bfloat16
    cnt = _deg(dstp, ones1d, zeros1d)
    g1, dinv_rep = _prep(cnt.reshape(NP, 1), x_pad)
    s1 = _agg2(g1.reshape(2 * NP, 128), src2, dstp, zeros2d)
    g2 = _layer(s1, g1, dinv_rep, W1.astype(bf16), b1, Cin=2)
    s2 = _agg4(g2.reshape(4 * NP, 128), src4, dstp, zeros2d)
    g3 = _layer(s2, g2, dinv_rep, W2.astype(bf16), b2, Cin=4)
    s3 = _agg4(g3.reshape(4 * NP, 128), src4, dstp, zeros2d)
    out = _head(s3, g3, dinv_rep, W3.astype(bf16), b3, Wout.astype(bf16), bout)
    return out[:N]


# speedup vs baseline: 1.2196x; 1.0026x over previous
import functools

import jax
import jax.numpy as jnp
from jax import lax
from jax.experimental import pallas as pl
from jax.experimental.pallas import tpu as pltpu
from jax.experimental.pallas import tpu_sc as plsc

N = 10000
NP = 10240
E = 160000
NB = 80
EPT = NB * 128
EP = 16 * EPT
RPT = NP // 16
T = 512
NT = NP // T

_f32 = jnp.float32


def _sc_mesh():
    return plsc.VectorSubcoreMesh(
        core_axis_name="c", subcore_axis_name="s", num_cores=2, num_subcores=16
    )


def _make_deg():
    @functools.partial(
        pl.kernel,
        out_type=jax.ShapeDtypeStruct((NP,), _f32),
        mesh=_sc_mesh(),
        scratch_types=[
            pltpu.VMEM((NB, 128), jnp.int32),
            pltpu.VMEM((128,), _f32),
            pltpu.VMEM_SHARED((NP,), _f32),
        ],
    )
    def deg_kernel(dst_hbm, ones_hbm, zeros1_hbm, cnt_hbm, dst_v, ones_v, acc_sh):
        c = lax.axis_index("c")
        s = lax.axis_index("s")
        pltpu.sync_copy(dst_hbm.at[s], dst_v)
        pltpu.sync_copy(ones_hbm, ones_v)
        pltpu.sync_copy(zeros1_hbm, acc_sh.at[pl.ds(s * RPT, RPT)])
        plsc.subcore_barrier()

        def body(b, carry):
            pltpu.sync_copy(ones_v, acc_sh.at[dst_v.at[b]], add=True)
            return carry

        lax.fori_loop(0, NB, body, 0)
        plsc.subcore_barrier()

        @pl.when(c == 0)
        def _():
            pltpu.sync_copy(
                acc_sh.at[pl.ds(s * RPT, RPT)], cnt_hbm.at[pl.ds(s * RPT, RPT)]
            )

    return deg_kernel


def _make_agg(C):
    CC = C // 2

    NG = NB // 16

    if True:
        @functools.partial(
            pl.kernel,
            out_type=jax.ShapeDtypeStruct((C, NP, 128), _f32),
            mesh=_sc_mesh(),
            scratch_types=[
                pltpu.VMEM((2, 16, 128), jnp.int32),
                pltpu.VMEM((NB, 128), jnp.int32),
                pltpu.VMEM((2, 128, 128), _f32),
                pltpu.VMEM_SHARED((NP, 128), _f32),
                pltpu.SemaphoreType.DMA,
                pltpu.SemaphoreType.DMA,
                pltpu.SemaphoreType.DMA,
            ],
        )
        def agg_kernel(g_hbm, srcall_hbm, dst_hbm, zeros_hbm, s_hbm,
                       src_g, dst_v, rows_v, acc_sh, sem_a, sem_b, sem_s):
            c = lax.axis_index("c")
            s = lax.axis_index("s")
            pltpu.sync_copy(dst_hbm.at[s], dst_v)
            pltpu.sync_copy(zeros_hbm, acc_sh.at[pl.ds(s * RPT, RPT)])
            plsc.subcore_barrier()
            for j in range(CC):
                cc = c * CC + j
                pltpu.sync_copy(srcall_hbm.at[cc, s, pl.ds(0, 16)], src_g.at[0])

                def group_body(g, carry):
                    gp = g % 2

                    @pl.when(g < NG - 1)
                    def _():
                        pltpu.async_copy(
                            srcall_hbm.at[cc, s, pl.ds((g + 1) * 16, 16)],
                            src_g.at[1 - gp], sem_s)

                    def pair(i, carry2):
                        b0 = 2 * i
                        b1 = b0 + 1
                        base = g * 16
                        pltpu.async_copy(
                            g_hbm.at[src_g.at[gp, b1]], rows_v.at[1], sem_b)
                        pltpu.make_async_copy(
                            g_hbm.at[src_g.at[gp, b0]], rows_v.at[0],
                            sem_a).wait()
                        pltpu.sync_copy(
                            rows_v.at[0], acc_sh.at[dst_v.at[base + b0]],
                            add=True)

                        @pl.when(i < 7)
                        def _():
                            pltpu.async_copy(
                                g_hbm.at[src_g.at[gp, b0 + 2]], rows_v.at[0],
                                sem_a)

                        pltpu.make_async_copy(
                            g_hbm.at[src_g.at[gp, b1]], rows_v.at[1],
                            sem_b).wait()
                        pltpu.sync_copy(
                            rows_v.at[1], acc_sh.at[dst_v.at[base + b1]],
                            add=True)
                        return carry2

                    lax.fori_loop(0, 8, pair, 0)

                    @pl.when(g < NG - 1)
                    def _():
                        pltpu.make_async_copy(
                            srcall_hbm.at[cc, s, pl.ds((g + 1) * 16, 16)],
                            src_g.at[1 - gp], sem_s).wait()
                        pltpu.async_copy(
                            g_hbm.at[src_g.at[1 - gp, 0]], rows_v.at[0], sem_a)

                    return carry

                pltpu.async_copy(g_hbm.at[src_g.at[0, 0]], rows_v.at[0], sem_a)
                lax.fori_loop(0, NG, group_body, 0)
                plsc.subcore_barrier()
                pltpu.sync_copy(
                    acc_sh.at[pl.ds(s * RPT, RPT)],
                    s_hbm.at[cc, pl.ds(s * RPT, RPT)],
                )
                if j + 1 < CC:
                    pltpu.sync_copy(zeros_hbm, acc_sh.at[pl.ds(s * RPT, RPT)])
                    plsc.subcore_barrier()

    return agg_kernel


_deg = _make_deg()
_agg2 = _make_agg(2)
_agg4 = _make_agg(4)


def _prep(cnt2d, x_pad):
    def body(cnt_ref, x_ref, g_ref, dr_ref):
        dinv = lax.rsqrt(cnt_ref[...] + 1.0)
        dinv_rep = jnp.broadcast_to(dinv, (T, 128))
        g_ref[0] = x_ref[...] * dinv_rep

        @pl.when(pl.program_id(1) == 0)
        def _():
            dr_ref[...] = dinv_rep

    return pl.pallas_call(
        body,
        grid=(NT, 2),
        in_specs=[
            pl.BlockSpec((T, 1), lambda i, c: (i, 0)),
            pl.BlockSpec((T, 128), lambda i, c: (i, c)),
        ],
        out_specs=[
            pl.BlockSpec((1, T, 128), lambda i, c: (c, i, 0)),
            pl.BlockSpec((T, 128), lambda i, c: (i, 0)),
        ],
        out_shape=[
            jax.ShapeDtypeStruct((2, NP, 128), _f32),
            jax.ShapeDtypeStruct((NP, 128), _f32),
        ],
    )(cnt2d, x_pad)


def _layer(s_arr, g_arr, dinv_rep, W, b, Cin):
    Cout = 4

    def body(s_ref, g_ref, d_ref, w_ref, b_ref, o_ref):
        d = d_ref[...]
        acc = jnp.zeros((T, 512), _f32)
        for ci in range(Cin):
            u = ((s_ref[ci] + g_ref[ci]) * d).astype(jnp.bfloat16)
            acc += jnp.dot(u, w_ref[pl.ds(ci * 128, 128), :],
                           preferred_element_type=_f32)
        x = jnp.maximum(acc + b_ref[...], 0.0)
        for co in range(Cout):
            o_ref[co] = x[:, co * 128:(co + 1) * 128] * d

    return pl.pallas_call(
        body,
        grid=(NT,),
        in_specs=[
            pl.BlockSpec((Cin, T, 128), lambda i: (0, i, 0)),
            pl.BlockSpec((Cin, T, 128), lambda i: (0, i, 0)),
            pl.BlockSpec((T, 128), lambda i: (i, 0)),
            pl.BlockSpec((Cin * 128, 512), lambda i: (0, 0)),
            pl.BlockSpec((1, 512), lambda i: (0, 0)),
        ],
        out_specs=pl.BlockSpec((Cout, T, 128), lambda i: (0, i, 0)),
        out_shape=jax.ShapeDtypeStruct((Cout, NP, 128), _f32),
    )(s_arr, g_arr, dinv_rep, W, b.reshape(1, 512))


def _head(s_arr, g_arr, dinv_rep, W3, b3, Wout, bout):
    def body(s_ref, g_ref, d_ref, w3_ref, b3_ref, wo_ref, bo_ref, o_ref):
        d = d_ref[...]
        acc = jnp.zeros((T, 512), _f32)
        for ci in range(4):
            u = ((s_ref[ci] + g_ref[ci]) * d).astype(jnp.bfloat16)
            acc += jnp.dot(u, w3_ref[pl.ds(ci * 128, 128), :],
                           preferred_element_type=_f32)
        x4 = jnp.maximum(acc + b3_ref[...], 0.0).astype(jnp.bfloat16)
        o_ref[...] = jnp.dot(x4, wo_ref[...], preferred_element_type=_f32) + bo_ref[...]

    return pl.pallas_call(
        body,
        grid=(NT,),
        in_specs=[
            pl.BlockSpec((4, T, 128), lambda i: (0, i, 0)),
            pl.BlockSpec((4, T, 128), lambda i: (0, i, 0)),
            pl.BlockSpec((T, 128), lambda i: (i, 0)),
            pl.BlockSpec((512, 512), lambda i: (0, 0)),
            pl.BlockSpec((1, 512), lambda i: (0, 0)),
            pl.BlockSpec((512, 512), lambda i: (0, 0)),
            pl.BlockSpec((1, 512), lambda i: (0, 0)),
        ],
        out_specs=pl.BlockSpec((T, 512), lambda i: (i, 0)),
        out_shape=jax.ShapeDtypeStruct((NP, 512), _f32),
    )(s_arr, g_arr, dinv_rep, W3, b3.reshape(1, 512), Wout, bout.reshape(1, 512))


def kernel(x_node, edge_index, W1, b1, W2, b2, W3, b3, Wout, bout):
    src = edge_index[0].astype(jnp.int32)
    dst = edge_index[1].astype(jnp.int32)
    pad = EP - E
    pad_src = (jnp.arange(pad, dtype=jnp.int32) * 97) % N
    pad_dst = N + jnp.arange(pad, dtype=jnp.int32) % (NP - N)
    srcp = jnp.concatenate([src, pad_src]).reshape(16, EPT)
    dstp = jnp.concatenate([dst, pad_dst]).reshape(16, NB, 128)
    src2 = (srcp[None] + (jnp.arange(2, dtype=jnp.int32) * NP)[:, None, None]
            ).reshape(2, 16, NB, 128)
    src4 = (srcp[None] + (jnp.arange(4, dtype=jnp.int32) * NP)[:, None, None]
            ).reshape(4, 16, NB, 128)
    zeros2d = jnp.zeros((RPT, 128), _f32)
    zeros1d = jnp.zeros((RPT,), _f32)
    ones1d = jnp.ones((128,), _f32)
    x_pad = jnp.pad(x_node, ((0, NP - N), (0, 0)))

    bf16 = jnp.bfloat16
    cnt = _deg(dstp, ones1d, zeros1d)
    g1, dinv_rep = _prep(cnt.reshape(NP, 1), x_pad)
    s1 = _agg2(g1.reshape(2 * NP, 128), src2, dstp, zeros2d)
    g2 = _layer(s1, g1, dinv_rep, W1.astype(bf16), b1, Cin=2)
    s2 = _agg4(g2.reshape(4 * NP, 128), src4, dstp, zeros2d)
    g3 = _layer(s2, g2, dinv_rep, W2.astype(bf16), b2, Cin=4)
    s3 = _agg4(g3.reshape(4 * NP, 128), src4, dstp, zeros2d)
    out = _head(s3, g3, dinv_rep, W3.astype(bf16), b3, Wout.astype(bf16), bout)
    return out[:N]


# speedup vs baseline: 1.2438x; 1.0198x over previous
import functools

import jax
import jax.numpy as jnp
from jax import lax
from jax.experimental import pallas as pl
from jax.experimental.pallas import tpu as pltpu
from jax.experimental.pallas import tpu_sc as plsc

N = 10000
NP = 10240
E = 160000
NB = 80
EPT = NB * 128
EP = 16 * EPT
RPT = NP // 16
T = 512
NT = NP // T

_f32 = jnp.float32


def _sc_mesh():
    return plsc.VectorSubcoreMesh(
        core_axis_name="c", subcore_axis_name="s", num_cores=2, num_subcores=16
    )


def _make_deg():
    @functools.partial(
        pl.kernel,
        out_type=jax.ShapeDtypeStruct((NP,), _f32),
        mesh=_sc_mesh(),
        scratch_types=[
            pltpu.VMEM((NB, 128), jnp.int32),
            pltpu.VMEM((128,), _f32),
            pltpu.VMEM_SHARED((NP,), _f32),
        ],
    )
    def deg_kernel(dst_hbm, ones_hbm, zeros1_hbm, cnt_hbm, dst_v, ones_v, acc_sh):
        c = lax.axis_index("c")
        s = lax.axis_index("s")
        pltpu.sync_copy(dst_hbm.at[s], dst_v)
        pltpu.sync_copy(ones_hbm, ones_v)
        pltpu.sync_copy(zeros1_hbm, acc_sh.at[pl.ds(s * RPT, RPT)])
        plsc.subcore_barrier()

        def body(b, carry):
            pltpu.sync_copy(ones_v, acc_sh.at[dst_v.at[b]], add=True)
            return carry

        lax.fori_loop(0, NB, body, 0)
        plsc.subcore_barrier()

        @pl.when(c == 0)
        def _():
            pltpu.sync_copy(
                acc_sh.at[pl.ds(s * RPT, RPT)], cnt_hbm.at[pl.ds(s * RPT, RPT)]
            )

    return deg_kernel


def _make_agg(C):
    CC = C // 2

    NG = NB // 16

    if True:
        @functools.partial(
            pl.kernel,
            out_type=jax.ShapeDtypeStruct((C, NP, 128), _f32),
            mesh=_sc_mesh(),
            scratch_types=[
                pltpu.VMEM((2, 16, 128), jnp.int32),
                pltpu.VMEM((NB, 128), jnp.int32),
                pltpu.VMEM((2, 128, 128), _f32),
                pltpu.VMEM_SHARED((NP, 128), _f32),
                pltpu.SemaphoreType.DMA,
                pltpu.SemaphoreType.DMA,
                pltpu.SemaphoreType.DMA,
            ],
        )
        def agg_kernel(g_hbm, srcall_hbm, dst_hbm, zeros_hbm, s_hbm,
                       src_g, dst_v, rows_v, acc_sh, sem_a, sem_b, sem_s):
            c = lax.axis_index("c")
            s = lax.axis_index("s")
            pltpu.sync_copy(dst_hbm.at[s], dst_v)
            pltpu.sync_copy(zeros_hbm, acc_sh.at[pl.ds(s * RPT, RPT)])
            plsc.subcore_barrier()
            for j in range(CC):
                cc = c * CC + j
                pltpu.sync_copy(srcall_hbm.at[cc, s, pl.ds(0, 16)], src_g.at[0])

                def group_body(g, carry):
                    gp = g % 2

                    @pl.when(g < NG - 1)
                    def _():
                        pltpu.async_copy(
                            srcall_hbm.at[cc, s, pl.ds((g + 1) * 16, 16)],
                            src_g.at[1 - gp], sem_s)

                    def pair(i, carry2):
                        b0 = 2 * i
                        b1 = b0 + 1
                        base = g * 16
                        pltpu.async_copy(
                            g_hbm.at[src_g.at[gp, b1]], rows_v.at[1], sem_b)
                        pltpu.make_async_copy(
                            g_hbm.at[src_g.at[gp, b0]], rows_v.at[0],
                            sem_a).wait()
                        pltpu.sync_copy(
                            rows_v.at[0], acc_sh.at[dst_v.at[base + b0]],
                            add=True)

                        @pl.when(i < 7)
                        def _():
                            pltpu.async_copy(
                                g_hbm.at[src_g.at[gp, b0 + 2]], rows_v.at[0],
                                sem_a)

                        pltpu.make_async_copy(
                            g_hbm.at[src_g.at[gp, b1]], rows_v.at[1],
                            sem_b).wait()
                        pltpu.sync_copy(
                            rows_v.at[1], acc_sh.at[dst_v.at[base + b1]],
                            add=True)
                        return carry2

                    lax.fori_loop(0, 8, pair, 0)

                    @pl.when(g < NG - 1)
                    def _():
                        pltpu.make_async_copy(
                            srcall_hbm.at[cc, s, pl.ds((g + 1) * 16, 16)],
                            src_g.at[1 - gp], sem_s).wait()
                        pltpu.async_copy(
                            g_hbm.at[src_g.at[1 - gp, 0]], rows_v.at[0], sem_a)

                    return carry

                pltpu.async_copy(g_hbm.at[src_g.at[0, 0]], rows_v.at[0], sem_a)
                lax.fori_loop(0, NG, group_body, 0)
                plsc.subcore_barrier()
                pltpu.sync_copy(
                    acc_sh.at[pl.ds(s * RPT, RPT)],
                    s_hbm.at[cc, pl.ds(s * RPT, RPT)],
                )
                if j + 1 < CC:
                    pltpu.sync_copy(zeros_hbm, acc_sh.at[pl.ds(s * RPT, RPT)])
                    plsc.subcore_barrier()

    return agg_kernel


_deg = _make_deg()
_agg2 = _make_agg(2)
_agg4 = _make_agg(4)


def _prep(cnt2d, x_pad):
    def body(cnt_ref, x_ref, g_ref, dr_ref):
        dinv = lax.rsqrt(cnt_ref[...] + 1.0)
        dinv_rep = jnp.broadcast_to(dinv, (T, 128))
        g_ref[0] = x_ref[...] * dinv_rep

        @pl.when(pl.program_id(1) == 0)
        def _():
            dr_ref[...] = dinv_rep

    return pl.pallas_call(
        body,
        grid=(NT, 2),
        in_specs=[
            pl.BlockSpec((T, 1), lambda i, c: (i, 0)),
            pl.BlockSpec((T, 128), lambda i, c: (i, c)),
        ],
        out_specs=[
            pl.BlockSpec((1, T, 128), lambda i, c: (c, i, 0)),
            pl.BlockSpec((T, 128), lambda i, c: (i, 0)),
        ],
        out_shape=[
            jax.ShapeDtypeStruct((2, NP, 128), _f32),
            jax.ShapeDtypeStruct((NP, 128), _f32),
        ],
    )(cnt2d, x_pad)


def _layer(s_arr, g_arr, dinv_rep, W, b, Cin):
    Cout = 4

    def body(s_ref, g_ref, d_ref, w_ref, b_ref, o_ref):
        d = d_ref[...]
        acc = jnp.zeros((T, 512), _f32)
        for ci in range(Cin):
            u = ((s_ref[ci] + g_ref[ci]) * d).astype(jnp.bfloat16)
            acc += jnp.dot(u, w_ref[pl.ds(ci * 128, 128), :],
                           preferred_element_type=_f32)
        x = jnp.maximum(acc + b_ref[...], 0.0)
        for co in range(Cout):
            o_ref[co] = x[:, co * 128:(co + 1) * 128] * d

    return pl.pallas_call(
        body,
        grid=(NT,),
        in_specs=[
            pl.BlockSpec((Cin, T, 128), lambda i: (0, i, 0)),
            pl.BlockSpec((Cin, T, 128), lambda i: (0, i, 0)),
            pl.BlockSpec((T, 128), lambda i: (i, 0)),
            pl.BlockSpec((Cin * 128, 512), lambda i: (0, 0)),
            pl.BlockSpec((1, 512), lambda i: (0, 0)),
        ],
        out_specs=pl.BlockSpec((Cout, T, 128), lambda i: (0, i, 0)),
        out_shape=jax.ShapeDtypeStruct((Cout, NP, 128), _f32),
    )(s_arr, g_arr, dinv_rep, W, b.reshape(1, 512))


def _head(s_arr, g_arr, dinv_rep, W3, b3, Wout, bout):
    def body(s_ref, g_ref, d_ref, w3_ref, b3_ref, wo_ref, bo_ref, o_ref):
        d = d_ref[...]
        acc = jnp.zeros((T, 512), _f32)
        for ci in range(4):
            u = ((s_ref[ci] + g_ref[ci]) * d).astype(jnp.bfloat16)
            acc += jnp.dot(u, w3_ref[pl.ds(ci * 128, 128), :],
                           preferred_element_type=_f32)
        x4 = jnp.maximum(acc + b3_ref[...], 0.0).astype(jnp.bfloat16)
        o_ref[...] = jnp.dot(x4, wo_ref[...], preferred_element_type=_f32) + bo_ref[...]

    return pl.pallas_call(
        body,
        grid=(NT,),
        in_specs=[
            pl.BlockSpec((4, T, 128), lambda i: (0, i, 0)),
            pl.BlockSpec((4, T, 128), lambda i: (0, i, 0)),
            pl.BlockSpec((T, 128), lambda i: (i, 0)),
            pl.BlockSpec((512, 512), lambda i: (0, 0)),
            pl.BlockSpec((1, 512), lambda i: (0, 0)),
            pl.BlockSpec((512, 512), lambda i: (0, 0)),
            pl.BlockSpec((1, 512), lambda i: (0, 0)),
        ],
        out_specs=pl.BlockSpec((T, 512), lambda i: (i, 0)),
        out_shape=jax.ShapeDtypeStruct((N, 512), _f32),
    )(s_arr, g_arr, dinv_rep, W3, b3.reshape(1, 512), Wout, bout.reshape(1, 512))


def kernel(x_node, edge_index, W1, b1, W2, b2, W3, b3, Wout, bout):
    src = edge_index[0].astype(jnp.int32)
    dst = edge_index[1].astype(jnp.int32)
    pad = EP - E
    pad_src = (jnp.arange(pad, dtype=jnp.int32) * 97) % N
    pad_dst = N + jnp.arange(pad, dtype=jnp.int32) % (NP - N)
    srcp = jnp.concatenate([src, pad_src]).reshape(16, EPT)
    dstp = jnp.concatenate([dst, pad_dst]).reshape(16, NB, 128)
    src2 = (srcp[None] + (jnp.arange(2, dtype=jnp.int32) * NP)[:, None, None]
            ).reshape(2, 16, NB, 128)
    src4 = (srcp[None] + (jnp.arange(4, dtype=jnp.int32) * NP)[:, None, None]
            ).reshape(4, 16, NB, 128)
    zeros2d = jnp.zeros((RPT, 128), _f32)
    zeros1d = jnp.zeros((RPT,), _f32)
    ones1d = jnp.ones((128,), _f32)
    x_pad = jnp.pad(x_node, ((0, NP - N), (0, 0)))

    bf16 = jnp.bfloat16
    cnt = _deg(dstp, ones1d, zeros1d)
    g1, dinv_rep = _prep(cnt.reshape(NP, 1), x_pad)
    s1 = _agg2(g1.reshape(2 * NP, 128), src2, dstp, zeros2d)
    g2 = _layer(s1, g1, dinv_rep, W1.astype(bf16), b1, Cin=2)
    s2 = _agg4(g2.reshape(4 * NP, 128), src4, dstp, zeros2d)
    g3 = _layer(s2, g2, dinv_rep, W2.astype(bf16), b2, Cin=4)
    s3 = _agg4(g3.reshape(4 * NP, 128), src4, dstp, zeros2d)
    return _head(s3, g3, dinv_rep, W3.astype(bf16), b3, Wout.astype(bf16), bout)


# speedup vs baseline: 1.2687x; 1.0200x over previous
import functools

import jax
import jax.numpy as jnp
from jax import lax
from jax.experimental import pallas as pl
from jax.experimental.pallas import tpu as pltpu
from jax.experimental.pallas import tpu_sc as plsc

N = 10000
NP = 10240
E = 160000
NB = 80
EPT = NB * 128
EP = 16 * EPT
RPT = NP // 16
T = 512
NT = NP // T

_f32 = jnp.float32


def _sc_mesh():
    return plsc.VectorSubcoreMesh(
        core_axis_name="c", subcore_axis_name="s", num_cores=2, num_subcores=16
    )


def _make_deg():
    @functools.partial(
        pl.kernel,
        out_type=jax.ShapeDtypeStruct((NP,), _f32),
        mesh=_sc_mesh(),
        scratch_types=[
            pltpu.VMEM((NB, 128), jnp.int32),
            pltpu.VMEM((128,), _f32),
            pltpu.VMEM_SHARED((NP,), _f32),
        ],
    )
    def deg_kernel(dst_hbm, ones_hbm, zeros1_hbm, cnt_hbm, dst_v, ones_v, acc_sh):
        c = lax.axis_index("c")
        s = lax.axis_index("s")
        pltpu.sync_copy(dst_hbm.at[s], dst_v)
        pltpu.sync_copy(ones_hbm, ones_v)
        pltpu.sync_copy(zeros1_hbm, acc_sh.at[pl.ds(s * RPT, RPT)])
        plsc.subcore_barrier()

        def body(b, carry):
            pltpu.sync_copy(ones_v, acc_sh.at[dst_v.at[b]], add=True)
            return carry

        lax.fori_loop(0, NB, body, 0)
        plsc.subcore_barrier()

        @pl.when(c == 0)
        def _():
            pltpu.sync_copy(
                acc_sh.at[pl.ds(s * RPT, RPT)], cnt_hbm.at[pl.ds(s * RPT, RPT)]
            )

    return deg_kernel


def _make_agg(C):
    CC = C // 2

    NG = NB // 16

    if True:
        @functools.partial(
            pl.kernel,
            out_type=jax.ShapeDtypeStruct((C, NP, 128), _f32),
            mesh=_sc_mesh(),
            scratch_types=[
                pltpu.VMEM((2, 16, 128), jnp.int32),
                pltpu.VMEM((NB, 128), jnp.int32),
                pltpu.VMEM((2, 128, 128), _f32),
                pltpu.VMEM_SHARED((NP, 128), _f32),
                pltpu.SemaphoreType.DMA,
                pltpu.SemaphoreType.DMA,
                pltpu.SemaphoreType.DMA,
            ],
        )
        def agg_kernel(g_hbm, srcall_hbm, dst_hbm, s_hbm,
                       src_g, dst_v, rows_v, acc_sh, sem_a, sem_b, sem_s):
            c = lax.axis_index("c")
            s = lax.axis_index("s")
            pltpu.sync_copy(dst_hbm.at[s], dst_v)
            for j in range(CC):
                cc = c * CC + j
                pltpu.sync_copy(
                    g_hbm.at[pl.ds(cc * NP + s * RPT, RPT)],
                    acc_sh.at[pl.ds(s * RPT, RPT)])
                pltpu.sync_copy(srcall_hbm.at[cc, s, pl.ds(0, 16)], src_g.at[0])
                plsc.subcore_barrier()

                def group_body(g, carry):
                    gp = g % 2

                    @pl.when(g < NG - 1)
                    def _():
                        pltpu.async_copy(
                            srcall_hbm.at[cc, s, pl.ds((g + 1) * 16, 16)],
                            src_g.at[1 - gp], sem_s)

                    def pair(i, carry2):
                        b0 = 2 * i
                        b1 = b0 + 1
                        base = g * 16
                        pltpu.async_copy(
                            g_hbm.at[src_g.at[gp, b1]], rows_v.at[1], sem_b)
                        pltpu.make_async_copy(
                            g_hbm.at[src_g.at[gp, b0]], rows_v.at[0],
                            sem_a).wait()
                        pltpu.sync_copy(
                            rows_v.at[0], acc_sh.at[dst_v.at[base + b0]],
                            add=True)

                        @pl.when(i < 7)
                        def _():
                            pltpu.async_copy(
                                g_hbm.at[src_g.at[gp, b0 + 2]], rows_v.at[0],
                                sem_a)

                        pltpu.make_async_copy(
                            g_hbm.at[src_g.at[gp, b1]], rows_v.at[1],
                            sem_b).wait()
                        pltpu.sync_copy(
                            rows_v.at[1], acc_sh.at[dst_v.at[base + b1]],
                            add=True)
                        return carry2

                    lax.fori_loop(0, 8, pair, 0)

                    @pl.when(g < NG - 1)
                    def _():
                        pltpu.make_async_copy(
                            srcall_hbm.at[cc, s, pl.ds((g + 1) * 16, 16)],
                            src_g.at[1 - gp], sem_s).wait()
                        pltpu.async_copy(
                            g_hbm.at[src_g.at[1 - gp, 0]], rows_v.at[0], sem_a)

                    return carry

                pltpu.async_copy(g_hbm.at[src_g.at[0, 0]], rows_v.at[0], sem_a)
                lax.fori_loop(0, NG, group_body, 0)
                plsc.subcore_barrier()
                pltpu.sync_copy(
                    acc_sh.at[pl.ds(s * RPT, RPT)],
                    s_hbm.at[cc, pl.ds(s * RPT, RPT)],
                )

    return agg_kernel


_deg = _make_deg()
_agg2 = _make_agg(2)
_agg4 = _make_agg(4)


def _prep(cnt2d, x_pad):
    def body(cnt_ref, x_ref, g_ref, dr_ref):
        dinv = lax.rsqrt(cnt_ref[...] + 1.0)
        dinv_rep = jnp.broadcast_to(dinv, (T, 128))
        g_ref[0] = x_ref[...] * dinv_rep

        @pl.when(pl.program_id(1) == 0)
        def _():
            dr_ref[...] = dinv_rep

    return pl.pallas_call(
        body,
        grid=(NT, 2),
        in_specs=[
            pl.BlockSpec((T, 1), lambda i, c: (i, 0)),
            pl.BlockSpec((T, 128), lambda i, c: (i, c)),
        ],
        out_specs=[
            pl.BlockSpec((1, T, 128), lambda i, c: (c, i, 0)),
            pl.BlockSpec((T, 128), lambda i, c: (i, 0)),
        ],
        out_shape=[
            jax.ShapeDtypeStruct((2, NP, 128), _f32),
            jax.ShapeDtypeStruct((NP, 128), _f32),
        ],
    )(cnt2d, x_pad)


def _layer(s_arr, dinv_rep, W, b, Cin):
    Cout = 4

    def body(s_ref, d_ref, w_ref, b_ref, o_ref):
        d = d_ref[...]
        acc = jnp.zeros((T, 512), _f32)
        for ci in range(Cin):
            u = (s_ref[ci] * d).astype(jnp.bfloat16)
            acc += jnp.dot(u, w_ref[pl.ds(ci * 128, 128), :],
                           preferred_element_type=_f32)
        x = jnp.maximum(acc + b_ref[...], 0.0)
        for co in range(Cout):
            o_ref[co] = x[:, co * 128:(co + 1) * 128] * d

    return pl.pallas_call(
        body,
        grid=(NT,),
        in_specs=[
            pl.BlockSpec((Cin, T, 128), lambda i: (0, i, 0)),
            pl.BlockSpec((T, 128), lambda i: (i, 0)),
            pl.BlockSpec((Cin * 128, 512), lambda i: (0, 0)),
            pl.BlockSpec((1, 512), lambda i: (0, 0)),
        ],
        out_specs=pl.BlockSpec((Cout, T, 128), lambda i: (0, i, 0)),
        out_shape=jax.ShapeDtypeStruct((Cout, NP, 128), _f32),
    )(s_arr, dinv_rep, W, b.reshape(1, 512))


def _head(s_arr, dinv_rep, W3, b3, Wout, bout):
    def body(s_ref, d_ref, w3_ref, b3_ref, wo_ref, bo_ref, o_ref):
        d = d_ref[...]
        acc = jnp.zeros((T, 512), _f32)
        for ci in range(4):
            u = (s_ref[ci] * d).astype(jnp.bfloat16)
            acc += jnp.dot(u, w3_ref[pl.ds(ci * 128, 128), :],
                           preferred_element_type=_f32)
        x4 = jnp.maximum(acc + b3_ref[...], 0.0).astype(jnp.bfloat16)
        o_ref[...] = jnp.dot(x4, wo_ref[...], preferred_element_type=_f32) + bo_ref[...]

    return pl.pallas_call(
        body,
        grid=(NT,),
        in_specs=[
            pl.BlockSpec((4, T, 128), lambda i: (0, i, 0)),
            pl.BlockSpec((T, 128), lambda i: (i, 0)),
            pl.BlockSpec((512, 512), lambda i: (0, 0)),
            pl.BlockSpec((1, 512), lambda i: (0, 0)),
            pl.BlockSpec((512, 512), lambda i: (0, 0)),
            pl.BlockSpec((1, 512), lambda i: (0, 0)),
        ],
        out_specs=pl.BlockSpec((T, 512), lambda i: (i, 0)),
        out_shape=jax.ShapeDtypeStruct((N, 512), _f32),
    )(s_arr, dinv_rep, W3, b3.reshape(1, 512), Wout, bout.reshape(1, 512))


def kernel(x_node, edge_index, W1, b1, W2, b2, W3, b3, Wout, bout):
    src = edge_index[0].astype(jnp.int32)
    dst = edge_index[1].astype(jnp.int32)
    pad = EP - E
    pad_src = (jnp.arange(pad, dtype=jnp.int32) * 97) % N
    pad_dst = N + jnp.arange(pad, dtype=jnp.int32) % (NP - N)
    srcp = jnp.concatenate([src, pad_src]).reshape(16, EPT)
    dstp = jnp.concatenate([dst, pad_dst]).reshape(16, NB, 128)
    src2 = (srcp[None] + (jnp.arange(2, dtype=jnp.int32) * NP)[:, None, None]
            ).reshape(2, 16, NB, 128)
    src4 = (srcp[None] + (jnp.arange(4, dtype=jnp.int32) * NP)[:, None, None]
            ).reshape(4, 16, NB, 128)
    zeros1d = jnp.zeros((RPT,), _f32)
    ones1d = jnp.ones((128,), _f32)
    x_pad = jnp.pad(x_node, ((0, NP - N), (0, 0)))

    bf16 = jnp.bfloat16
    cnt = _deg(dstp, ones1d, zeros1d)
    g1, dinv_rep = _prep(cnt.reshape(NP, 1), x_pad)
    s1 = _agg2(g1.reshape(2 * NP, 128), src2, dstp)
    g2 = _layer(s1, dinv_rep, W1.astype(bf16), b1, Cin=2)
    s2 = _agg4(g2.reshape(4 * NP, 128), src4, dstp)
    g3 = _layer(s2, dinv_rep, W2.astype(bf16), b2, Cin=4)
    s3 = _agg4(g3.reshape(4 * NP, 128), src4, dstp)
    return _head(s3, dinv_rep, W3.astype(bf16), b3, Wout.astype(bf16), bout)
